# baseline (speedup 1.0000x reference)
import math

import jax
import jax.numpy as jnp
from jax import lax
from jax.experimental import pallas as pl
from jax.experimental.pallas import tpu as pltpu
from jax.experimental.pallas import tpu_sc as plsc

DIM = 64
SCALE = math.sqrt(DIM)
LANES = 16
CHUNK = 128


def _make_kernel(num_workers: int, steps: int):
    total_rows = num_workers * steps * CHUNK
    mesh = plsc.VectorSubcoreMesh(core_axis_name="c", subcore_axis_name="s")

    def body(idx_hbm, table_hbm, out_hbm, idx_v, rows_v, sem):
        nc = mesh.num_cores
        wid = lax.axis_index("s") * nc + lax.axis_index("c")
        base = wid * (steps * CHUNK)
        pltpu.sync_copy(idx_hbm.at[wid], idx_v)

        @pl.loop(0, steps)
        def _(j):
            pltpu.async_copy(table_hbm.at[idx_v.at[j]], rows_v, sem).wait()
            @pl.loop(0, CHUNK)
            def _(r):
                for k in range(DIM // LANES):
                    sl = pl.ds(k * LANES, LANES)
                    rows_v[r, sl] = rows_v[r, sl] * SCALE
            pltpu.sync_copy(rows_v, out_hbm.at[pl.ds(base + j * CHUNK, CHUNK)])

    kern = pl.kernel(
        body,
        out_type=jax.ShapeDtypeStruct((total_rows, DIM), jnp.float32),
        mesh=mesh,
        compiler_params=pltpu.CompilerParams(use_tc_tiling_on_sc=False),
        scratch_types=[
            pltpu.VMEM((steps, CHUNK), jnp.int32),
            pltpu.VMEM((CHUNK, DIM), jnp.float32),
            pltpu.SemaphoreType.DMA,
        ],
    )
    return kern


def kernel(x, w_embed):
    batch, hist = x.shape
    total = batch * hist
    info = plsc.get_sparse_core_info()
    num_workers = info.num_cores * info.num_subcores
    steps = total // (num_workers * CHUNK)
    assert steps * num_workers * CHUNK == total
    idx = x.reshape(num_workers, steps, CHUNK).astype(jnp.int32)
    out = _make_kernel(num_workers, steps)(idx, w_embed)
    return out.reshape(batch, hist, DIM)
